# baseline (speedup 1.0000x reference)
import functools

import jax
import jax.numpy as jnp
from jax import lax
from jax.experimental import pallas as pl
from jax.experimental.pallas import tpu as pltpu
from jax.experimental.pallas import tpu_sc as plsc

B = 16384
EMB = 16
NC = 2
NS = 16
L = 16
NW = NC * NS
BPW = B // NW
NCH = 8
CH = BPW // NCH
VOCAB = 1000000
EPG = 128 // EMB
GRAN = VOCAB // EPG
NFB = VOCAB // 128
TAIL_V = NFB * 128
LIN_ROWS = -(-VOCAB // 128)

_MESH = plsc.VectorSubcoreMesh(core_axis_name="c", subcore_axis_name="s")
_CPARAMS = pltpu.CompilerParams(needs_layout_passes=False)


def _shuffle(in_v, out_v, blocks):
    for sb in range(blocks):
        for dt in range(2):
            for j in range(8):
                for v16 in range(8):
                    vals = in_v[dt * 8 + j, pl.ds(sb * 128 + v16 * L, L)]
                    out_v[sb * L + v16 * 2 + dt, pl.ds(j * L, L)] = vals


@functools.partial(
    pl.kernel,
    mesh=_MESH,
    compiler_params=_CPARAMS,
    out_type=[
        jax.ShapeDtypeStruct((GRAN, 128), jnp.float32),
        jax.ShapeDtypeStruct((GRAN, 128), jnp.float32),
    ],
    scratch_types=[
        pltpu.VMEM((EMB, 512), jnp.float32), pltpu.VMEM((EMB, 512), jnp.float32),
        pltpu.VMEM((EMB, 512), jnp.float32), pltpu.VMEM((EMB, 512), jnp.float32),
        pltpu.VMEM((EMB, 512), jnp.float32), pltpu.VMEM((EMB, 512), jnp.float32),
        pltpu.VMEM((64, 128), jnp.float32), pltpu.VMEM((64, 128), jnp.float32),
        pltpu.VMEM((64, 128), jnp.float32), pltpu.VMEM((64, 128), jnp.float32),
        pltpu.VMEM((64, 128), jnp.float32), pltpu.VMEM((64, 128), jnp.float32),
        pltpu.SemaphoreType.DMA, pltpu.SemaphoreType.DMA,
        pltpu.SemaphoreType.DMA, pltpu.SemaphoreType.DMA,
        pltpu.SemaphoreType.DMA, pltpu.SemaphoreType.DMA,
    ],
)
def _relayout(uT_hbm, iT_hbm, utail_hbm, itail_hbm, uC_hbm, iC_hbm,
              inu0, inu1, inu2, ini0, ini1, ini2,
              outu0, outu1, outu2, outi0, outi1, outi2,
              sin0, sin1, sin2, sout0, sout1, sout2):
    wid = lax.axis_index("s") * NC + lax.axis_index("c")
    inu = (inu0, inu1, inu2)
    ini = (ini0, ini1, ini2)
    outu = (outu0, outu1, outu2)
    outi = (outi0, outi1, outi2)
    sin = (sin0, sin1, sin2)
    sout = (sout0, sout1, sout2)

    base = wid * 244 + jnp.minimum(wid, 4)

    def bstart(m):
        return jnp.minimum(base + 4 * m, NFB - 4)

    def start_in(m, p):
        src = pl.ds(bstart(m) * 128, 512)
        pltpu.async_copy(uT_hbm.at[:, src], inu[p], sin[p])
        pltpu.async_copy(iT_hbm.at[:, src], ini[p], sin[p])

    def drain(buf_u, buf_i, sem):
        pltpu.make_async_copy(uT_hbm.at[:, pl.ds(0, 512)], buf_u, sem).wait()
        pltpu.make_async_copy(iT_hbm.at[:, pl.ds(0, 512)], buf_i, sem).wait()

    def drain_out(p):
        pltpu.make_async_copy(uC_hbm.at[pl.ds(0, 64)], outu[p], sout[p]).wait()
        pltpu.make_async_copy(iC_hbm.at[pl.ds(0, 64)], outi[p], sout[p]).wait()

    for p in range(3):
        start_in(p, p)

    def body(j, carry):
        for p in range(3):
            m = 3 * j + p
            drain(inu[p], ini[p], sin[p])

            @pl.when(j > 0)
            def _():
                drain_out(p)
            _shuffle(inu[p], outu[p], 4)
            _shuffle(ini[p], outi[p], 4)

            @pl.when(j < 20)
            def _():
                start_in(m + 3, p)

            dst = pl.ds(bstart(m) * 16, 64)
            pltpu.async_copy(outu[p], uC_hbm.at[dst], sout[p])
            pltpu.async_copy(outi[p], iC_hbm.at[dst], sout[p])
        return carry

    lax.fori_loop(0, 21, body, 0)
    drain_out(0)
    drain_out(1)
    drain_out(2)

    @pl.when(wid == NW - 1)
    def _():
        dst = pl.ds(NFB * 16, 8)
        pltpu.sync_copy(utail_hbm, outu0.at[pl.ds(0, 8)])
        pltpu.sync_copy(itail_hbm, outi0.at[pl.ds(0, 8)])
        pltpu.sync_copy(outu0.at[pl.ds(0, 8)], uC_hbm.at[dst])
        pltpu.sync_copy(outi0.at[pl.ds(0, 8)], iC_hbm.at[dst])


@functools.partial(
    pl.kernel,
    mesh=_MESH,
    compiler_params=_CPARAMS,
    out_type=[
        jax.ShapeDtypeStruct((NW, BPW), jnp.float32),
        jax.ShapeDtypeStruct((NW, L), jnp.float32),
    ],
    scratch_types=[
        pltpu.VMEM((BPW,), jnp.int32),
        pltpu.VMEM((BPW,), jnp.int32),
        pltpu.VMEM((BPW,), jnp.int32),
        pltpu.VMEM((BPW,), jnp.int32),
        pltpu.VMEM((BPW,), jnp.int32),
        pltpu.VMEM((BPW,), jnp.int32),
        pltpu.VMEM((BPW,), jnp.int32),
        pltpu.VMEM((BPW,), jnp.int32),
        pltpu.VMEM((2, CH, 128), jnp.float32),
        pltpu.VMEM((2, CH, 128), jnp.float32),
        pltpu.VMEM((2, CH, 128), jnp.float32),
        pltpu.VMEM((2, CH, 128), jnp.float32),
        pltpu.VMEM((2, CH, 128), jnp.float32),
        pltpu.VMEM((2, CH, 128), jnp.float32),
        pltpu.VMEM((BPW,), jnp.float32),
        pltpu.VMEM((L,), jnp.float32),
        pltpu.SemaphoreType.DMA,
        pltpu.SemaphoreType.DMA,
    ],
)
def _fm_sc(users_hbm, items_hbm, uemb_hbm, iemb_hbm, ulin_hbm, ilin_hbm,
           out_hbm, aux_hbm,
           uidx_v, iidx_v, ueg_v, ueg1_v, ieg_v, ieg1_v, ulg_v, ilg_v,
           ue0_v, ue1_v, ie0_v, ie1_v, ul_v, il_v, out_v, acc_v, sem_a, sem_b):
    wid = lax.axis_index("s") * NC + lax.axis_index("c")

    pltpu.sync_copy(users_hbm.at[wid], uidx_v)
    pltpu.sync_copy(items_hbm.at[wid], iidx_v)

    for i in range(BPW // L):
        s = pl.ds(i * L, L)
        u = uidx_v[s]
        t = iidx_v[s]
        ug = (u >> 4) << 1
        ig = (t >> 4) << 1
        ueg_v[s] = ug
        ueg1_v[s] = ug + 1
        ieg_v[s] = ig
        ieg1_v[s] = ig + 1
        ulg_v[s] = u >> 7
        ilg_v[s] = t >> 7

    def start(ch):
        p = ch % 2
        sem = sem_a if p == 0 else sem_b
        rows = pl.ds(ch * CH, CH)
        return [
            pltpu.async_copy(uemb_hbm.at[ueg_v.at[rows]], ue0_v.at[p], sem),
            pltpu.async_copy(uemb_hbm.at[ueg1_v.at[rows]], ue1_v.at[p], sem),
            pltpu.async_copy(iemb_hbm.at[ieg_v.at[rows]], ie0_v.at[p], sem),
            pltpu.async_copy(iemb_hbm.at[ieg1_v.at[rows]], ie1_v.at[p], sem),
            pltpu.async_copy(ulin_hbm.at[ulg_v.at[rows]], ul_v.at[p], sem),
            pltpu.async_copy(ilin_hbm.at[ilg_v.at[rows]], il_v.at[p], sem),
        ]

    acc = jnp.zeros((L,), jnp.float32)
    inflight = start(0)
    for ch in range(NCH):
        for cp in inflight:
            cp.wait()
        if ch + 1 < NCH:
            inflight = start(ch + 1)
        p = ch % 2
        pp = jnp.full((L,), p, jnp.int32)
        for g in range(CH // L):
            rloc = g * L + lax.iota(jnp.int32, L)
            s = pl.ds(ch * CH + g * L, L)
            uid = uidx_v[s]
            iid = iidx_v[s]
            uoff = uid & (L - 1)
            ioff = iid & (L - 1)
            fm = jnp.zeros((L,), jnp.float32)
            for d in range(EMB):
                ubuf = ue0_v if d < 8 else ue1_v
                ibuf = ie0_v if d < 8 else ie1_v
                uc = plsc.load_gather(ubuf, [pp, rloc, uoff + (d & 7) * L])
                ic = plsc.load_gather(ibuf, [pp, rloc, ioff + (d & 7) * L])
                fm = fm + uc * ic
            ul = plsc.load_gather(ul_v, [pp, rloc, uid & 127])
            il = plsc.load_gather(il_v, [pp, rloc, iid & 127])
            x = ul + il + fm
            sig = 1.0 / (1.0 + jnp.exp(-x))
            out_v[s] = sig
            acc = acc + fm * fm

    acc_v[...] = acc
    pltpu.sync_copy(out_v, out_hbm.at[wid])
    pltpu.sync_copy(acc_v, aux_hbm.at[wid])


def kernel(users, items, user_emb, item_emb, user_lin, item_lin):
    u = users.reshape(NW, BPW).astype(jnp.int32)
    i = items.reshape(NW, BPW).astype(jnp.int32)
    utail = user_emb[TAIL_V:].reshape(4, L, 2, 8).transpose(0, 2, 3, 1).reshape(8, 128)
    itail = item_emb[TAIL_V:].reshape(4, L, 2, 8).transpose(0, 2, 3, 1).reshape(8, 128)
    uemb, iemb = _relayout(user_emb.T, item_emb.T, utail, itail)
    pad = LIN_ROWS * 128 - VOCAB
    ulin = jnp.pad(user_lin.reshape(-1), (0, pad)).reshape(LIN_ROWS, 128)
    ilin = jnp.pad(item_lin.reshape(-1), (0, pad)).reshape(LIN_ROWS, 128)
    sig, parts = _fm_sc(u, i, uemb, iemb, ulin, ilin)
    aux = 0.1 * (jnp.sum(parts) / B)
    return (sig.reshape(B, 1), aux)


# speedup vs baseline: 1.0409x; 1.0409x over previous
import functools

import jax
import jax.numpy as jnp
from jax import lax
from jax.experimental import pallas as pl
from jax.experimental.pallas import tpu as pltpu
from jax.experimental.pallas import tpu_sc as plsc

B = 16384
EMB = 16
NC = 2
NS = 16
L = 16
NW = NC * NS
BPW = B // NW
NCH = 8
CH = BPW // NCH
VOCAB = 1000000
EPG = 128 // EMB
GRAN = VOCAB // EPG
NFB = VOCAB // 128
TAIL_V = NFB * 128
LIN_ROWS = -(-VOCAB // 128)

_MESH = plsc.VectorSubcoreMesh(core_axis_name="c", subcore_axis_name="s")
_CPARAMS = pltpu.CompilerParams(needs_layout_passes=False)


def _shuffle(in_v, out_v, blocks):
    for sb in range(blocks):
        for dt in range(2):
            for j in range(8):
                for v16 in range(8):
                    vals = in_v[dt * 8 + j, pl.ds(sb * 128 + v16 * L, L)]
                    out_v[sb * L + v16 * 2 + dt, pl.ds(j * L, L)] = vals


@functools.partial(
    pl.kernel,
    mesh=_MESH,
    compiler_params=_CPARAMS,
    out_type=[
        jax.ShapeDtypeStruct((GRAN, 128), jnp.float32),
        jax.ShapeDtypeStruct((GRAN, 128), jnp.float32),
    ],
    scratch_types=[
        pltpu.VMEM((EMB, 256), jnp.float32), pltpu.VMEM((EMB, 256), jnp.float32),
        pltpu.VMEM((EMB, 256), jnp.float32), pltpu.VMEM((EMB, 256), jnp.float32),
        pltpu.VMEM((EMB, 256), jnp.float32), pltpu.VMEM((EMB, 256), jnp.float32),
        pltpu.VMEM((EMB, 256), jnp.float32), pltpu.VMEM((EMB, 256), jnp.float32),
        pltpu.VMEM((32, 128), jnp.float32), pltpu.VMEM((32, 128), jnp.float32),
        pltpu.VMEM((32, 128), jnp.float32), pltpu.VMEM((32, 128), jnp.float32),
        pltpu.VMEM((32, 128), jnp.float32), pltpu.VMEM((32, 128), jnp.float32),
        pltpu.VMEM((32, 128), jnp.float32), pltpu.VMEM((32, 128), jnp.float32),
        pltpu.SemaphoreType.DMA, pltpu.SemaphoreType.DMA,
        pltpu.SemaphoreType.DMA, pltpu.SemaphoreType.DMA,
        pltpu.SemaphoreType.DMA, pltpu.SemaphoreType.DMA,
        pltpu.SemaphoreType.DMA, pltpu.SemaphoreType.DMA,
    ],
)
def _relayout(uT_hbm, iT_hbm, utail_hbm, itail_hbm, uC_hbm, iC_hbm,
              inu0, inu1, inu2, inu3, ini0, ini1, ini2, ini3,
              outu0, outu1, outu2, outu3, outi0, outi1, outi2, outi3,
              sin0, sin1, sin2, sin3, sout0, sout1, sout2, sout3):
    wid = lax.axis_index("s") * NC + lax.axis_index("c")
    inu = (inu0, inu1, inu2, inu3)
    ini = (ini0, ini1, ini2, ini3)
    outu = (outu0, outu1, outu2, outu3)
    outi = (outi0, outi1, outi2, outi3)
    sin = (sin0, sin1, sin2, sin3)
    sout = (sout0, sout1, sout2, sout3)

    base = wid * 244 + jnp.minimum(wid, 4)

    def bstart(m):
        return jnp.minimum(base + 2 * m, NFB - 2)

    def start_in(m, p):
        src = pl.ds(bstart(m) * 128, 256)
        pltpu.async_copy(uT_hbm.at[:, src], inu[p], sin[p])
        pltpu.async_copy(iT_hbm.at[:, src], ini[p], sin[p])

    def drain(buf_u, buf_i, sem):
        pltpu.make_async_copy(uT_hbm.at[:, pl.ds(0, 256)], buf_u, sem).wait()
        pltpu.make_async_copy(iT_hbm.at[:, pl.ds(0, 256)], buf_i, sem).wait()

    def drain_out(p):
        pltpu.make_async_copy(uC_hbm.at[pl.ds(0, 32)], outu[p], sout[p]).wait()
        pltpu.make_async_copy(iC_hbm.at[pl.ds(0, 32)], outi[p], sout[p]).wait()

    for p in range(4):
        start_in(p, p)

    def body(j, carry):
        for p in range(4):
            m = 4 * j + p
            drain(inu[p], ini[p], sin[p])

            @pl.when(j > 0)
            def _():
                drain_out(p)
            _shuffle(inu[p], outu[p], 2)
            _shuffle(ini[p], outi[p], 2)

            @pl.when(j < 30)
            def _():
                start_in(m + 4, p)

            dst = pl.ds(bstart(m) * 16, 32)
            pltpu.async_copy(outu[p], uC_hbm.at[dst], sout[p])
            pltpu.async_copy(outi[p], iC_hbm.at[dst], sout[p])
        return carry

    lax.fori_loop(0, 31, body, 0)
    for p in range(4):
        drain_out(p)

    @pl.when(wid == NW - 1)
    def _():
        dst = pl.ds(NFB * 16, 8)
        pltpu.sync_copy(utail_hbm, outu0.at[pl.ds(0, 8)])
        pltpu.sync_copy(itail_hbm, outi0.at[pl.ds(0, 8)])
        pltpu.sync_copy(outu0.at[pl.ds(0, 8)], uC_hbm.at[dst])
        pltpu.sync_copy(outi0.at[pl.ds(0, 8)], iC_hbm.at[dst])


@functools.partial(
    pl.kernel,
    mesh=_MESH,
    compiler_params=_CPARAMS,
    out_type=[
        jax.ShapeDtypeStruct((NW, BPW), jnp.float32),
        jax.ShapeDtypeStruct((NW, L), jnp.float32),
    ],
    scratch_types=[
        pltpu.VMEM((BPW,), jnp.int32),
        pltpu.VMEM((BPW,), jnp.int32),
        pltpu.VMEM((BPW,), jnp.int32),
        pltpu.VMEM((BPW,), jnp.int32),
        pltpu.VMEM((BPW,), jnp.int32),
        pltpu.VMEM((BPW,), jnp.int32),
        pltpu.VMEM((BPW,), jnp.int32),
        pltpu.VMEM((BPW,), jnp.int32),
        pltpu.VMEM((2, CH, 128), jnp.float32),
        pltpu.VMEM((2, CH, 128), jnp.float32),
        pltpu.VMEM((2, CH, 128), jnp.float32),
        pltpu.VMEM((2, CH, 128), jnp.float32),
        pltpu.VMEM((2, CH, 128), jnp.float32),
        pltpu.VMEM((2, CH, 128), jnp.float32),
        pltpu.VMEM((BPW,), jnp.float32),
        pltpu.VMEM((L,), jnp.float32),
        pltpu.SemaphoreType.DMA,
        pltpu.SemaphoreType.DMA,
    ],
)
def _fm_sc(users_hbm, items_hbm, uemb_hbm, iemb_hbm, ulin_hbm, ilin_hbm,
           out_hbm, aux_hbm,
           uidx_v, iidx_v, ueg_v, ueg1_v, ieg_v, ieg1_v, ulg_v, ilg_v,
           ue0_v, ue1_v, ie0_v, ie1_v, ul_v, il_v, out_v, acc_v, sem_a, sem_b):
    wid = lax.axis_index("s") * NC + lax.axis_index("c")

    pltpu.sync_copy(users_hbm.at[wid], uidx_v)
    pltpu.sync_copy(items_hbm.at[wid], iidx_v)

    for i in range(BPW // L):
        s = pl.ds(i * L, L)
        u = uidx_v[s]
        t = iidx_v[s]
        ug = (u >> 4) << 1
        ig = (t >> 4) << 1
        ueg_v[s] = ug
        ueg1_v[s] = ug + 1
        ieg_v[s] = ig
        ieg1_v[s] = ig + 1
        ulg_v[s] = u >> 7
        ilg_v[s] = t >> 7

    def start(ch):
        p = ch % 2
        sem = sem_a if p == 0 else sem_b
        rows = pl.ds(ch * CH, CH)
        return [
            pltpu.async_copy(uemb_hbm.at[ueg_v.at[rows]], ue0_v.at[p], sem),
            pltpu.async_copy(uemb_hbm.at[ueg1_v.at[rows]], ue1_v.at[p], sem),
            pltpu.async_copy(iemb_hbm.at[ieg_v.at[rows]], ie0_v.at[p], sem),
            pltpu.async_copy(iemb_hbm.at[ieg1_v.at[rows]], ie1_v.at[p], sem),
            pltpu.async_copy(ulin_hbm.at[ulg_v.at[rows]], ul_v.at[p], sem),
            pltpu.async_copy(ilin_hbm.at[ilg_v.at[rows]], il_v.at[p], sem),
        ]

    acc = jnp.zeros((L,), jnp.float32)
    inflight = start(0)
    for ch in range(NCH):
        for cp in inflight:
            cp.wait()
        if ch + 1 < NCH:
            inflight = start(ch + 1)
        p = ch % 2
        pp = jnp.full((L,), p, jnp.int32)
        for g in range(CH // L):
            rloc = g * L + lax.iota(jnp.int32, L)
            s = pl.ds(ch * CH + g * L, L)
            uid = uidx_v[s]
            iid = iidx_v[s]
            uoff = uid & (L - 1)
            ioff = iid & (L - 1)
            fm = jnp.zeros((L,), jnp.float32)
            for d in range(EMB):
                ubuf = ue0_v if d < 8 else ue1_v
                ibuf = ie0_v if d < 8 else ie1_v
                uc = plsc.load_gather(ubuf, [pp, rloc, uoff + (d & 7) * L])
                ic = plsc.load_gather(ibuf, [pp, rloc, ioff + (d & 7) * L])
                fm = fm + uc * ic
            ul = plsc.load_gather(ul_v, [pp, rloc, uid & 127])
            il = plsc.load_gather(il_v, [pp, rloc, iid & 127])
            x = ul + il + fm
            sig = 1.0 / (1.0 + jnp.exp(-x))
            out_v[s] = sig
            acc = acc + fm * fm

    acc_v[...] = acc
    pltpu.sync_copy(out_v, out_hbm.at[wid])
    pltpu.sync_copy(acc_v, aux_hbm.at[wid])


def kernel(users, items, user_emb, item_emb, user_lin, item_lin):
    u = users.reshape(NW, BPW).astype(jnp.int32)
    i = items.reshape(NW, BPW).astype(jnp.int32)
    utail = user_emb[TAIL_V:].reshape(4, L, 2, 8).transpose(0, 2, 3, 1).reshape(8, 128)
    itail = item_emb[TAIL_V:].reshape(4, L, 2, 8).transpose(0, 2, 3, 1).reshape(8, 128)
    uemb, iemb = _relayout(user_emb.T, item_emb.T, utail, itail)
    pad = LIN_ROWS * 128 - VOCAB
    ulin = jnp.pad(user_lin.reshape(-1), (0, pad)).reshape(LIN_ROWS, 128)
    ilin = jnp.pad(item_lin.reshape(-1), (0, pad)).reshape(LIN_ROWS, 128)
    sig, parts = _fm_sc(u, i, uemb, iemb, ulin, ilin)
    aux = 0.1 * (jnp.sum(parts) / B)
    return (sig.reshape(B, 1), aux)


# speedup vs baseline: 1.0818x; 1.0393x over previous
import functools

import jax
import jax.numpy as jnp
from jax import lax
from jax.experimental import pallas as pl
from jax.experimental.pallas import tpu as pltpu
from jax.experimental.pallas import tpu_sc as plsc

B = 16384
EMB = 16
NC = 2
NS = 16
L = 16
NW = NC * NS
BPW = B // NW
NCH = 8
CH = BPW // NCH
VOCAB = 1000000
EPG = 128 // EMB
GRAN = VOCAB // EPG
NFB = VOCAB // 128
TAIL_V = NFB * 128
LIN_ROWS = -(-VOCAB // 128)

_MESH = plsc.VectorSubcoreMesh(core_axis_name="c", subcore_axis_name="s")
_CPARAMS = pltpu.CompilerParams(needs_layout_passes=False)


def _shuffle(in_v, out_v, blocks):
    for sb in range(blocks):
        for dt in range(2):
            for j in range(8):
                for v16 in range(8):
                    vals = in_v[dt * 8 + j, pl.ds(sb * 128 + v16 * L, L)]
                    out_v[sb * L + v16 * 2 + dt, pl.ds(j * L, L)] = vals


@functools.partial(
    pl.kernel,
    mesh=_MESH,
    compiler_params=_CPARAMS,
    out_type=[
        jax.ShapeDtypeStruct((GRAN, 128), jnp.float32),
        jax.ShapeDtypeStruct((GRAN, 128), jnp.float32),
    ],
    scratch_types=[
        pltpu.VMEM((EMB, 256), jnp.float32), pltpu.VMEM((EMB, 256), jnp.float32),
        pltpu.VMEM((EMB, 256), jnp.float32), pltpu.VMEM((EMB, 256), jnp.float32),
        pltpu.VMEM((EMB, 256), jnp.float32), pltpu.VMEM((EMB, 256), jnp.float32),
        pltpu.VMEM((EMB, 256), jnp.float32), pltpu.VMEM((EMB, 256), jnp.float32),
        pltpu.VMEM((32, 128), jnp.float32), pltpu.VMEM((32, 128), jnp.float32),
        pltpu.VMEM((32, 128), jnp.float32), pltpu.VMEM((32, 128), jnp.float32),
        pltpu.VMEM((32, 128), jnp.float32), pltpu.VMEM((32, 128), jnp.float32),
        pltpu.VMEM((32, 128), jnp.float32), pltpu.VMEM((32, 128), jnp.float32),
        pltpu.SemaphoreType.DMA, pltpu.SemaphoreType.DMA,
        pltpu.SemaphoreType.DMA, pltpu.SemaphoreType.DMA,
        pltpu.SemaphoreType.DMA, pltpu.SemaphoreType.DMA,
        pltpu.SemaphoreType.DMA, pltpu.SemaphoreType.DMA,
    ],
)
def _relayout(uT_hbm, iT_hbm, utail_hbm, itail_hbm, uC_hbm, iC_hbm,
              inu0, inu1, inu2, inu3, ini0, ini1, ini2, ini3,
              outu0, outu1, outu2, outu3, outi0, outi1, outi2, outi3,
              sin0, sin1, sin2, sin3, sout0, sout1, sout2, sout3):
    wid = lax.axis_index("s") * NC + lax.axis_index("c")
    inu = (inu0, inu1, inu2, inu3)
    ini = (ini0, ini1, ini2, ini3)
    outu = (outu0, outu1, outu2, outu3)
    outi = (outi0, outi1, outi2, outi3)
    sin = (sin0, sin1, sin2, sin3)
    sout = (sout0, sout1, sout2, sout3)

    base = wid * 244 + jnp.minimum(wid, 4)

    def bstart(m):
        return jnp.minimum(base + 2 * m, NFB - 2)

    def start_in(m, p):
        src = pl.ds(bstart(m) * 128, 256)
        pltpu.async_copy(uT_hbm.at[:, src], inu[p], sin[p])
        pltpu.async_copy(iT_hbm.at[:, src], ini[p], sin[p])

    def drain(buf_u, buf_i, sem):
        pltpu.make_async_copy(uT_hbm.at[:, pl.ds(0, 256)], buf_u, sem).wait()
        pltpu.make_async_copy(iT_hbm.at[:, pl.ds(0, 256)], buf_i, sem).wait()

    def drain_out(p):
        pltpu.make_async_copy(uC_hbm.at[pl.ds(0, 32)], outu[p], sout[p]).wait()
        pltpu.make_async_copy(iC_hbm.at[pl.ds(0, 32)], outi[p], sout[p]).wait()

    for p in range(4):
        start_in(p, p)

    def body(j, carry):
        for p in range(4):
            m = 4 * j + p
            drain(inu[p], ini[p], sin[p])

            @pl.when(j > 0)
            def _():
                drain_out(p)
            _shuffle(inu[p], outu[p], 2)
            _shuffle(ini[p], outi[p], 2)

            @pl.when(j < 30)
            def _():
                start_in(m + 4, p)

            dst = pl.ds(bstart(m) * 16, 32)
            pltpu.async_copy(outu[p], uC_hbm.at[dst], sout[p])
            pltpu.async_copy(outi[p], iC_hbm.at[dst], sout[p])
        return carry

    lax.fori_loop(0, 31, body, 0)
    for p in range(4):
        drain_out(p)

    @pl.when(wid == NW - 1)
    def _():
        dst = pl.ds(NFB * 16, 8)
        pltpu.sync_copy(utail_hbm, outu0.at[pl.ds(0, 8)])
        pltpu.sync_copy(itail_hbm, outi0.at[pl.ds(0, 8)])
        pltpu.sync_copy(outu0.at[pl.ds(0, 8)], uC_hbm.at[dst])
        pltpu.sync_copy(outi0.at[pl.ds(0, 8)], iC_hbm.at[dst])


@functools.partial(
    pl.kernel,
    mesh=_MESH,
    compiler_params=_CPARAMS,
    out_type=[
        jax.ShapeDtypeStruct((NW, BPW), jnp.float32),
        jax.ShapeDtypeStruct((NW, L), jnp.float32),
    ],
    scratch_types=[
        pltpu.VMEM((BPW,), jnp.int32),
        pltpu.VMEM((BPW,), jnp.int32),
        pltpu.VMEM((BPW,), jnp.int32),
        pltpu.VMEM((BPW,), jnp.int32),
        pltpu.VMEM((BPW,), jnp.int32),
        pltpu.VMEM((BPW,), jnp.int32),
        pltpu.VMEM((2, CH, 128), jnp.float32),
        pltpu.VMEM((2, CH, 128), jnp.float32),
        pltpu.VMEM((2, CH, 128), jnp.float32),
        pltpu.VMEM((2, CH, 128), jnp.float32),
        pltpu.VMEM((CH,), jnp.float32), pltpu.VMEM((CH,), jnp.float32),
        pltpu.VMEM((CH,), jnp.float32), pltpu.VMEM((CH,), jnp.float32),
        pltpu.VMEM((BPW,), jnp.float32),
        pltpu.VMEM((L,), jnp.float32),
        pltpu.SemaphoreType.DMA,
        pltpu.SemaphoreType.DMA,
    ],
)
def _fm_sc(users_hbm, items_hbm, uemb_hbm, iemb_hbm, ulin_hbm, ilin_hbm,
           out_hbm, aux_hbm,
           uidx_v, iidx_v, ueg_v, ueg1_v, ieg_v, ieg1_v,
           ue0_v, ue1_v, ie0_v, ie1_v, ulA, ulB, ilA, ilB,
           out_v, acc_v, sem_a, sem_b):
    wid = lax.axis_index("s") * NC + lax.axis_index("c")

    pltpu.sync_copy(users_hbm.at[wid], uidx_v)
    pltpu.sync_copy(items_hbm.at[wid], iidx_v)

    for i in range(BPW // L):
        s = pl.ds(i * L, L)
        u = uidx_v[s]
        t = iidx_v[s]
        ug = (u >> 4) << 1
        ig = (t >> 4) << 1
        ueg_v[s] = ug
        ueg1_v[s] = ug + 1
        ieg_v[s] = ig
        ieg1_v[s] = ig + 1

    def start(ch):
        p = ch % 2
        sem = sem_a if p == 0 else sem_b
        rows = pl.ds(ch * CH, CH)
        ul = ulA if p == 0 else ulB
        il = ilA if p == 0 else ilB
        return [
            pltpu.async_copy(uemb_hbm.at[ueg_v.at[rows]], ue0_v.at[p], sem),
            pltpu.async_copy(uemb_hbm.at[ueg1_v.at[rows]], ue1_v.at[p], sem),
            pltpu.async_copy(iemb_hbm.at[ieg_v.at[rows]], ie0_v.at[p], sem),
            pltpu.async_copy(iemb_hbm.at[ieg1_v.at[rows]], ie1_v.at[p], sem),
            pltpu.async_copy(ulin_hbm.at[uidx_v.at[rows]], ul, sem),
            pltpu.async_copy(ilin_hbm.at[iidx_v.at[rows]], il, sem),
        ]

    acc = jnp.zeros((L,), jnp.float32)
    inflight = start(0)
    for ch in range(NCH):
        for cp in inflight:
            cp.wait()
        if ch + 1 < NCH:
            inflight = start(ch + 1)
        p = ch % 2
        pp = jnp.full((L,), p, jnp.int32)
        for g in range(CH // L):
            rloc = g * L + lax.iota(jnp.int32, L)
            s = pl.ds(ch * CH + g * L, L)
            uid = uidx_v[s]
            iid = iidx_v[s]
            uoff = uid & (L - 1)
            ioff = iid & (L - 1)
            fm = jnp.zeros((L,), jnp.float32)
            for d in range(EMB):
                ubuf = ue0_v if d < 8 else ue1_v
                ibuf = ie0_v if d < 8 else ie1_v
                uc = plsc.load_gather(ubuf, [pp, rloc, uoff + (d & 7) * L])
                ic = plsc.load_gather(ibuf, [pp, rloc, ioff + (d & 7) * L])
                fm = fm + uc * ic
            sl = pl.ds(g * L, L)
            ul = (ulA if p == 0 else ulB)[sl]
            il = (ilA if p == 0 else ilB)[sl]
            x = ul + il + fm
            sig = 1.0 / (1.0 + jnp.exp(-x))
            out_v[s] = sig
            acc = acc + fm * fm

    acc_v[...] = acc
    pltpu.sync_copy(out_v, out_hbm.at[wid])
    pltpu.sync_copy(acc_v, aux_hbm.at[wid])


def kernel(users, items, user_emb, item_emb, user_lin, item_lin):
    u = users.reshape(NW, BPW).astype(jnp.int32)
    i = items.reshape(NW, BPW).astype(jnp.int32)
    utail = user_emb[TAIL_V:].reshape(4, L, 2, 8).transpose(0, 2, 3, 1).reshape(8, 128)
    itail = item_emb[TAIL_V:].reshape(4, L, 2, 8).transpose(0, 2, 3, 1).reshape(8, 128)
    uemb, iemb = _relayout(user_emb.T, item_emb.T, utail, itail)
    ulin = user_lin.reshape(-1)
    ilin = item_lin.reshape(-1)
    sig, parts = _fm_sc(u, i, uemb, iemb, ulin, ilin)
    aux = 0.1 * (jnp.sum(parts) / B)
    return (sig.reshape(B, 1), aux)


# speedup vs baseline: 1.0879x; 1.0057x over previous
import functools

import jax
import jax.numpy as jnp
from jax import lax
from jax.experimental import pallas as pl
from jax.experimental.pallas import tpu as pltpu
from jax.experimental.pallas import tpu_sc as plsc

B = 16384
EMB = 16
NC = 2
NS = 16
L = 16
NW = NC * NS
BPW = B // NW
NCH = 8
CH = BPW // NCH
VOCAB = 1000000
EPG = 128 // EMB
GRAN = VOCAB // EPG
NFB = VOCAB // 128
TAIL_V = NFB * 128

_MESH = plsc.VectorSubcoreMesh(core_axis_name="c", subcore_axis_name="s")
_CPARAMS = pltpu.CompilerParams(needs_layout_passes=False)


def _shuffle(in_v, out_v, blocks):
    for sb in range(blocks):
        for dt in range(2):
            for j in range(8):
                for v16 in range(8):
                    vals = in_v[dt * 8 + j, pl.ds(sb * 128 + v16 * L, L)]
                    out_v[sb * L + v16 * 2 + dt, pl.ds(j * L, L)] = vals


@functools.partial(
    pl.kernel,
    mesh=_MESH,
    compiler_params=_CPARAMS,
    out_type=[
        jax.ShapeDtypeStruct((GRAN, 128), jnp.float32),
        jax.ShapeDtypeStruct((GRAN, 128), jnp.float32),
    ],
    scratch_types=[
        pltpu.VMEM((EMB, 256), jnp.float32), pltpu.VMEM((EMB, 256), jnp.float32),
        pltpu.VMEM((EMB, 256), jnp.float32), pltpu.VMEM((EMB, 256), jnp.float32),
        pltpu.VMEM((EMB, 256), jnp.float32), pltpu.VMEM((EMB, 256), jnp.float32),
        pltpu.VMEM((EMB, 256), jnp.float32), pltpu.VMEM((EMB, 256), jnp.float32),
        pltpu.VMEM((32, 128), jnp.float32), pltpu.VMEM((32, 128), jnp.float32),
        pltpu.VMEM((32, 128), jnp.float32), pltpu.VMEM((32, 128), jnp.float32),
        pltpu.VMEM((32, 128), jnp.float32), pltpu.VMEM((32, 128), jnp.float32),
        pltpu.VMEM((32, 128), jnp.float32), pltpu.VMEM((32, 128), jnp.float32),
        pltpu.SemaphoreType.DMA, pltpu.SemaphoreType.DMA,
        pltpu.SemaphoreType.DMA, pltpu.SemaphoreType.DMA,
        pltpu.SemaphoreType.DMA, pltpu.SemaphoreType.DMA,
        pltpu.SemaphoreType.DMA, pltpu.SemaphoreType.DMA,
    ],
)
def _relayout(uT_hbm, iT_hbm, utail_hbm, itail_hbm, uC_hbm, iC_hbm,
              inu0, inu1, inu2, inu3, ini0, ini1, ini2, ini3,
              outu0, outu1, outu2, outu3, outi0, outi1, outi2, outi3,
              sin0, sin1, sin2, sin3, sout0, sout1, sout2, sout3):
    wid = lax.axis_index("s") * NC + lax.axis_index("c")
    inu = (inu0, inu1, inu2, inu3)
    ini = (ini0, ini1, ini2, ini3)
    outu = (outu0, outu1, outu2, outu3)
    outi = (outi0, outi1, outi2, outi3)
    sin = (sin0, sin1, sin2, sin3)
    sout = (sout0, sout1, sout2, sout3)

    base = wid * 244 + jnp.minimum(wid, 4)

    def bstart(m):
        return jnp.minimum(base + 2 * m, NFB - 2)

    def start_in(m, p):
        src = pl.ds(bstart(m) * 128, 256)
        pltpu.async_copy(uT_hbm.at[:, src], inu[p], sin[p])
        pltpu.async_copy(iT_hbm.at[:, src], ini[p], sin[p])

    def drain(buf_u, buf_i, sem):
        pltpu.make_async_copy(uT_hbm.at[:, pl.ds(0, 256)], buf_u, sem).wait()
        pltpu.make_async_copy(iT_hbm.at[:, pl.ds(0, 256)], buf_i, sem).wait()

    def drain_out(p):
        pltpu.make_async_copy(uC_hbm.at[pl.ds(0, 32)], outu[p], sout[p]).wait()
        pltpu.make_async_copy(iC_hbm.at[pl.ds(0, 32)], outi[p], sout[p]).wait()

    for p in range(4):
        start_in(p, p)

    def body(j, carry):
        for p in range(4):
            m = 4 * j + p
            drain(inu[p], ini[p], sin[p])

            @pl.when(j > 0)
            def _():
                drain_out(p)
            _shuffle(inu[p], outu[p], 2)
            _shuffle(ini[p], outi[p], 2)

            @pl.when(j < 30)
            def _():
                start_in(m + 4, p)

            dst = pl.ds(bstart(m) * 16, 32)
            pltpu.async_copy(outu[p], uC_hbm.at[dst], sout[p])
            pltpu.async_copy(outi[p], iC_hbm.at[dst], sout[p])
        return carry

    lax.fori_loop(0, 31, body, 0)
    for p in range(4):
        drain_out(p)

    @pl.when(wid == NW - 1)
    def _():
        dst = pl.ds(NFB * 16, 8)
        pltpu.sync_copy(utail_hbm, outu0.at[pl.ds(0, 8)])
        pltpu.sync_copy(itail_hbm, outi0.at[pl.ds(0, 8)])
        pltpu.sync_copy(outu0.at[pl.ds(0, 8)], uC_hbm.at[dst])
        pltpu.sync_copy(outi0.at[pl.ds(0, 8)], iC_hbm.at[dst])


@functools.partial(
    pl.kernel,
    mesh=_MESH,
    compiler_params=_CPARAMS,
    out_type=[
        jax.ShapeDtypeStruct((NW, BPW), jnp.float32),
        jax.ShapeDtypeStruct((NW, L), jnp.float32),
    ],
    scratch_types=[
        pltpu.VMEM((BPW,), jnp.int32),
        pltpu.VMEM((BPW,), jnp.int32),
        pltpu.VMEM((BPW,), jnp.int32),
        pltpu.VMEM((BPW,), jnp.int32),
        pltpu.VMEM((BPW,), jnp.int32),
        pltpu.VMEM((BPW,), jnp.int32),
        pltpu.VMEM((2, CH, 128), jnp.float32),
        pltpu.VMEM((2, CH, 128), jnp.float32),
        pltpu.VMEM((2, CH, 128), jnp.float32),
        pltpu.VMEM((2, CH, 128), jnp.float32),
        pltpu.VMEM((CH,), jnp.float32), pltpu.VMEM((CH,), jnp.float32),
        pltpu.VMEM((CH,), jnp.float32), pltpu.VMEM((CH,), jnp.float32),
        pltpu.VMEM((BPW,), jnp.float32),
        pltpu.VMEM((L,), jnp.float32),
        pltpu.SemaphoreType.DMA,
        pltpu.SemaphoreType.DMA,
    ],
)
def _fm_sc(users_hbm, items_hbm, uemb_hbm, iemb_hbm, ulin_hbm, ilin_hbm,
           out_hbm, aux_hbm,
           uidx_v, iidx_v, ueg_v, ueg1_v, ieg_v, ieg1_v,
           ue0_v, ue1_v, ie0_v, ie1_v, ulA, ulB, ilA, ilB,
           out_v, acc_v, sem_a, sem_b):
    wid = lax.axis_index("s") * NC + lax.axis_index("c")

    pltpu.sync_copy(users_hbm.at[wid], uidx_v)
    pltpu.sync_copy(items_hbm.at[wid], iidx_v)

    for i in range(BPW // L):
        s = pl.ds(i * L, L)
        u = uidx_v[s]
        t = iidx_v[s]
        ug = (u >> 4) << 1
        ig = (t >> 4) << 1
        ueg_v[s] = ug
        ueg1_v[s] = ug + 1
        ieg_v[s] = ig
        ieg1_v[s] = ig + 1

    def start(ch):
        p = ch % 2
        sem = sem_a if p == 0 else sem_b
        rows = pl.ds(ch * CH, CH)
        ul = ulA if p == 0 else ulB
        il = ilA if p == 0 else ilB
        return [
            pltpu.async_copy(uemb_hbm.at[ueg_v.at[rows]], ue0_v.at[p], sem),
            pltpu.async_copy(uemb_hbm.at[ueg1_v.at[rows]], ue1_v.at[p], sem),
            pltpu.async_copy(iemb_hbm.at[ieg_v.at[rows]], ie0_v.at[p], sem),
            pltpu.async_copy(iemb_hbm.at[ieg1_v.at[rows]], ie1_v.at[p], sem),
            pltpu.async_copy(ulin_hbm.at[uidx_v.at[rows]], ul, sem),
            pltpu.async_copy(ilin_hbm.at[iidx_v.at[rows]], il, sem),
        ]

    acc = jnp.zeros((L,), jnp.float32)
    inflight = start(0)
    for ch in range(NCH):
        for cp in inflight:
            cp.wait()
        if ch + 1 < NCH:
            inflight = start(ch + 1)
        p = ch % 2
        pp = jnp.full((L,), p, jnp.int32)
        for g in range(CH // L):
            rloc = g * L + lax.iota(jnp.int32, L)
            s = pl.ds(ch * CH + g * L, L)
            uid = uidx_v[s]
            iid = iidx_v[s]
            uoff = uid & (L - 1)
            ioff = iid & (L - 1)
            fm = jnp.zeros((L,), jnp.float32)
            for d in range(EMB):
                ubuf = ue0_v if d < 8 else ue1_v
                ibuf = ie0_v if d < 8 else ie1_v
                uc = plsc.load_gather(ubuf, [pp, rloc, uoff + (d & 7) * L])
                ic = plsc.load_gather(ibuf, [pp, rloc, ioff + (d & 7) * L])
                fm = fm + uc * ic
            sl = pl.ds(g * L, L)
            ul = (ulA if p == 0 else ulB)[sl]
            il = (ilA if p == 0 else ilB)[sl]
            x = ul + il + fm
            sig = 1.0 / (1.0 + jnp.exp(-x))
            out_v[s] = sig
            acc = acc + fm * fm

    acc_v[...] = acc
    pltpu.sync_copy(out_v, out_hbm.at[wid])
    pltpu.sync_copy(acc_v, aux_hbm.at[wid])


def kernel(users, items, user_emb, item_emb, user_lin, item_lin):
    u = users.reshape(NW, BPW).astype(jnp.int32)
    i = items.reshape(NW, BPW).astype(jnp.int32)
    utail = user_emb[TAIL_V:].reshape(4, L, 2, 8).transpose(0, 2, 3, 1).reshape(8, 128)
    itail = item_emb[TAIL_V:].reshape(4, L, 2, 8).transpose(0, 2, 3, 1).reshape(8, 128)
    uemb, iemb = _relayout(user_emb.T, item_emb.T, utail, itail)
    ulin = user_lin.reshape(-1)
    ilin = item_lin.reshape(-1)
    sig, parts = _fm_sc(u, i, uemb, iemb, ulin, ilin)
    aux = 0.1 * (jnp.sum(parts) / B)
    return (sig.reshape(B, 1), aux)
